# baseline (speedup 1.0000x reference)
import jax
import jax.numpy as jnp
from jax import lax
from jax.experimental import pallas as pl
from jax.experimental.pallas import tpu as pltpu

VOCAB = 100000
EMBED = 64
CTX = 50
HID = 128

VBLK = 2000
NSTREAM = 10
NBLK = VOCAB // VBLK
NSTEP = NBLK // NSTREAM


def _body(idx_ref, table_ref, w1_ref, b1_ref, *rest):
    w2_refs = rest[:NSTREAM]
    b2_ref, out_ref, sem, emb_ref, h_ref = rest[NSTREAM:]
    j = pl.program_id(0)

    @pl.when(j == 0)
    def _():
        copies = [
            pltpu.make_async_copy(
                table_ref.at[pl.ds(idx_ref[t], 1), :],
                emb_ref.at[pl.ds(t, 1), :],
                sem,
            )
            for t in range(CTX)
        ]
        for c in copies:
            c.start()
        for c in copies:
            c.wait()
        acc = jnp.zeros((1, HID), jnp.float32)
        for t in range(CTX):
            acc = acc + lax.dot_general(
                emb_ref[t:t + 1, :].astype(jnp.bfloat16),
                w1_ref[:, t * EMBED:(t + 1) * EMBED].astype(jnp.bfloat16),
                (((1,), (1,)), ((), ())), preferred_element_type=jnp.float32)
        h_ref[...] = jnp.maximum(acc + b1_ref[...], 0.0).astype(jnp.bfloat16)

    for g in range(NSTREAM):
        logits = lax.dot_general(
            h_ref[...], w2_refs[g][0].astype(jnp.bfloat16),
            (((1,), (1,)), ((), ())),
            preferred_element_type=jnp.float32,
        ) + b2_ref[:, 0, g * VBLK:(g + 1) * VBLK]
        out_ref[pl.ds(j * NSTREAM + g, 1)] = logits[None]

    @pl.when(j == NSTEP - 1)
    def _():
        x = out_ref[...]
        m = jnp.max(x)
        lse = m + jnp.log(jnp.sum(jnp.exp(x - m)))
        out_ref[...] = x - lse


def _w2_spec(g):
    return pl.BlockSpec((1, VBLK, HID), lambda j, g=g: (j * NSTREAM + g, 0, 0))


_call = pl.pallas_call(
    _body,
    grid=(NSTEP,),
    in_specs=[
        pl.BlockSpec(memory_space=pltpu.SMEM),
        pl.BlockSpec(memory_space=pl.ANY),
        pl.BlockSpec((HID, CTX * EMBED), lambda j: (0, 0)),
        pl.BlockSpec((1, HID), lambda j: (0, 0)),
    ] + [_w2_spec(g) for g in range(NSTREAM)] + [
        pl.BlockSpec((1, 1, NSTREAM * VBLK), lambda j: (j, 0, 0)),
    ],
    out_specs=pl.BlockSpec((NBLK, 1, VBLK), lambda j: (0, 0, 0)),
    out_shape=jax.ShapeDtypeStruct((NBLK, 1, VBLK), jnp.float32),
    scratch_shapes=[
        pltpu.SemaphoreType.DMA,
        pltpu.VMEM((CTX, EMBED), jnp.float32),
        pltpu.VMEM((1, HID), jnp.bfloat16),
    ],
    compiler_params=pltpu.CompilerParams(
        dimension_semantics=("arbitrary",)),
)


def kernel(inputs, emb_table, W1, b1, W2, b2):
    w2r = W2.reshape(NBLK, VBLK, HID)
    out = _call(
        inputs.astype(jnp.int32),
        emb_table,
        W1,
        b1.reshape(1, HID),
        *([w2r] * NSTREAM),
        b2.reshape(NSTEP, 1, NSTREAM * VBLK),
    )
    return out.reshape(1, VOCAB)
